# Initial kernel scaffold; baseline (speedup 1.0000x reference)
import jax
import jax.numpy as jnp
from jax.experimental import pallas as pl


def kernel(GO_adj, GO_weight, drug_smiles_fea, W1, b1, gamma1, beta1, W2, b2, gamma2, beta2):
    raise NotImplementedError("write your pallas kernel here")


# speedup vs baseline: 5.4842x; 5.4842x over previous
import functools

import jax
import jax.numpy as jnp
from jax import lax
from jax.experimental import pallas as pl
from jax.experimental.pallas import tpu as pltpu
from jax.experimental.pallas import tpu_sc as plsc

N = 10000
NP = 10240
D_IN = 256
D_HID = 256
D_OUT = 128
CHUNK = 128
RB = 640
NRB = NP // RB
RPT = NP // 16
EPS = 1e-5
SLOPE = 0.01

@functools.cache
def _mesh():
    return plsc.VectorSubcoreMesh(core_axis_name="c", subcore_axis_name="s")


@functools.partial(jax.jit, static_argnames=("e_pad",))
def _deg(col, ew, zeros_n, *, e_pad):
    ept = e_pad // 32
    nch = ept // CHUNK

    @functools.partial(
        pl.kernel,
        mesh=_mesh(),
        out_type=jax.ShapeDtypeStruct((2, NP), jnp.float32),
        scratch_types=[
            pltpu.VMEM((CHUNK,), jnp.int32),
            pltpu.VMEM((CHUNK,), jnp.float32),
            pltpu.VMEM_SHARED((NP,), jnp.float32),
        ],
    )
    def k(col_hbm, ew_hbm, zeros_hbm, out_hbm, col_v, ew_v, deg_sh):
        c = lax.axis_index("c")
        s = lax.axis_index("s")

        @pl.when(s == 0)
        def _():
            pltpu.sync_copy(zeros_hbm, deg_sh)

        plsc.subcore_barrier()
        base = c * (e_pad // 2) + s * ept

        def body(j, carry):
            off = base + j * CHUNK
            pltpu.sync_copy(col_hbm.at[pl.ds(off, CHUNK)], col_v)
            pltpu.sync_copy(ew_hbm.at[pl.ds(off, CHUNK)], ew_v)
            pltpu.sync_copy(ew_v, deg_sh.at[col_v], add=True)
            return carry

        lax.fori_loop(0, nch, body, 0)
        plsc.subcore_barrier()

        @pl.when(s == 0)
        def _():
            pltpu.sync_copy(deg_sh, out_hbm.at[c])

    return k(col, ew, zeros_n)


@functools.partial(jax.jit, static_argnames=("e_pad", "dh"))
def _scat_featsplit(rowst, col, ew, g, *, e_pad, dh):
    ept = e_pad // 16
    nch = ept // CHUNK
    ngrp = dh // 16

    @functools.partial(
        pl.kernel,
        mesh=_mesh(),
        out_type=jax.ShapeDtypeStruct((2 * NP, dh), jnp.float32),
        scratch_types=[
            pltpu.VMEM((CHUNK,), jnp.int32),
            pltpu.VMEM((CHUNK,), jnp.int32),
            pltpu.VMEM((CHUNK,), jnp.float32),
            pltpu.VMEM((CHUNK, dh), jnp.float32),
            pltpu.VMEM_SHARED((NP, dh), jnp.float32),
            pltpu.SemaphoreType.DMA,
        ],
    )
    def k(rowst_hbm, col_hbm, ew_hbm, g_hbm, out_hbm,
          idx_v, col_v, ew_v, rows_v, acc_sh, sem):
        c = lax.axis_index("c")
        s = lax.axis_index("s")
        pltpu.sync_copy(g_hbm.at[pl.ds(c * NP + s * RPT, RPT)],
                        acc_sh.at[pl.ds(s * RPT, RPT)])
        plsc.subcore_barrier()
        base = s * ept

        def body(j, carry):
            off = base + j * CHUNK
            pltpu.sync_copy(rowst_hbm.at[pl.ds(c * e_pad + off, CHUNK)], idx_v)
            pltpu.sync_copy(col_hbm.at[pl.ds(off, CHUNK)], col_v)
            pltpu.sync_copy(ew_hbm.at[pl.ds(off, CHUNK)], ew_v)
            pltpu.async_copy(g_hbm.at[idx_v], rows_v, sem).wait()

            def scale(g16, carry2):
                wvec = ew_v[pl.ds(g16 * 16, 16)]
                for lane in range(16):
                    w = wvec[lane]
                    kk = g16 * 16 + lane
                    for gi in range(ngrp):
                        sl = pl.ds(gi * 16, 16)
                        rows_v[kk, sl] = rows_v[kk, sl] * w
                return carry2

            lax.fori_loop(0, CHUNK // 16, scale, 0)
            pltpu.sync_copy(rows_v, acc_sh.at[col_v], add=True)
            return carry

        lax.fori_loop(0, nch, body, 0)
        plsc.subcore_barrier()
        pltpu.sync_copy(acc_sh.at[pl.ds(s * RPT, RPT)],
                        out_hbm.at[pl.ds(c * NP + s * RPT, RPT)])

    return k(rowst, col, ew, g)


@functools.partial(jax.jit, static_argnames=("e_pad", "dh"))
def _scat_edgesplit(row, col, ew, g, zeros_nd, *, e_pad, dh):
    ept = e_pad // 32
    nch = ept // CHUNK
    ngrp = dh // 16

    @functools.partial(
        pl.kernel,
        mesh=_mesh(),
        out_type=jax.ShapeDtypeStruct((2 * NP, dh), jnp.float32),
        scratch_types=[
            pltpu.VMEM((CHUNK,), jnp.int32),
            pltpu.VMEM((CHUNK,), jnp.int32),
            pltpu.VMEM((CHUNK,), jnp.float32),
            pltpu.VMEM((CHUNK, dh), jnp.float32),
            pltpu.VMEM_SHARED((NP, dh), jnp.float32),
            pltpu.SemaphoreType.DMA,
        ],
    )
    def k(row_hbm, col_hbm, ew_hbm, g_hbm, zeros_hbm, out_hbm,
          idx_v, col_v, ew_v, rows_v, acc_sh, sem):
        c = lax.axis_index("c")
        s = lax.axis_index("s")

        @pl.when(c == 0)
        def _():
            pltpu.sync_copy(g_hbm.at[pl.ds(s * RPT, RPT)],
                            acc_sh.at[pl.ds(s * RPT, RPT)])

        @pl.when(c != 0)
        def _():
            pltpu.sync_copy(zeros_hbm.at[pl.ds(s * RPT, RPT)],
                            acc_sh.at[pl.ds(s * RPT, RPT)])

        plsc.subcore_barrier()
        base = c * (e_pad // 2) + s * ept

        def body(j, carry):
            off = base + j * CHUNK
            pltpu.sync_copy(row_hbm.at[pl.ds(off, CHUNK)], idx_v)
            pltpu.sync_copy(col_hbm.at[pl.ds(off, CHUNK)], col_v)
            pltpu.sync_copy(ew_hbm.at[pl.ds(off, CHUNK)], ew_v)
            pltpu.async_copy(g_hbm.at[idx_v], rows_v, sem).wait()

            def scale(g16, carry2):
                wvec = ew_v[pl.ds(g16 * 16, 16)]
                for lane in range(16):
                    w = wvec[lane]
                    kk = g16 * 16 + lane
                    for gi in range(ngrp):
                        sl = pl.ds(gi * 16, 16)
                        rows_v[kk, sl] = rows_v[kk, sl] * w
                return carry2

            lax.fori_loop(0, CHUNK // 16, scale, 0)
            pltpu.sync_copy(rows_v, acc_sh.at[col_v], add=True)
            return carry

        lax.fori_loop(0, nch, body, 0)
        plsc.subcore_barrier()
        pltpu.sync_copy(acc_sh.at[pl.ds(s * RPT, RPT)],
                        out_hbm.at[pl.ds(c * NP + s * RPT, RPT)])

    return k(row, col, ew, g, zeros_nd)


def _dis_from(dp_ref):
    deg = 1.0 + dp_ref[:, 0:1] + dp_ref[:, 1:2]
    return lax.rsqrt(deg)


def _pre1_body(x_ref, w_ref, dp_ref, g_ref):
    dis = _dis_from(dp_ref)
    h = jnp.dot(x_ref[...], w_ref[...], preferred_element_type=jnp.float32)
    g_ref[...] = h * dis


def _pre1(x, w1, dpt):
    return pl.pallas_call(
        _pre1_body,
        grid=(2, NRB),
        in_specs=[
            pl.BlockSpec((RB, D_IN), lambda c, i: (i, 0)),
            pl.BlockSpec((D_IN, 128), lambda c, i: (0, c)),
            pl.BlockSpec((RB, 2), lambda c, i: (i, 0)),
        ],
        out_specs=pl.BlockSpec((RB, 128), lambda c, i: (c * NRB + i, 0)),
        out_shape=jax.ShapeDtypeStruct((2 * NP, 128), jnp.float32),
    )(x, w1, dpt)


def _stats1_body(acc_ref, dp_ref, gam_ref, bet_ref, scale_ref, shift_ref,
                 s_ref, q_ref):
    c = pl.program_id(0)
    i = pl.program_id(1)
    dis = _dis_from(dp_ref)
    v = acc_ref[...] * dis
    ps = jnp.sum(v, axis=0, keepdims=True)
    pq = jnp.sum(v * v, axis=0, keepdims=True)

    @pl.when(i == 0)
    def _():
        s_ref[...] = jnp.zeros_like(s_ref)
        q_ref[...] = jnp.zeros_like(q_ref)

    s_ref[...] += ps
    q_ref[...] += pq

    @pl.when(i == NRB - 1)
    def _():
        m = s_ref[...] * (1.0 / N)
        var = q_ref[...] * (1.0 / N) - m * m
        sc = gam_ref[pl.ds(c, 1), :] * lax.rsqrt(var + EPS)
        scale_ref[pl.ds(c, 1), :] = sc
        shift_ref[pl.ds(c, 1), :] = bet_ref[pl.ds(c, 1), :] - m * sc


def _stats1(acc_st, dpt, gam2, bet2):
    return pl.pallas_call(
        _stats1_body,
        grid=(2, NRB),
        in_specs=[
            pl.BlockSpec((RB, 128), lambda c, i: (c * NRB + i, 0)),
            pl.BlockSpec((RB, 2), lambda c, i: (i, 0)),
            pl.BlockSpec((2, 128), lambda c, i: (0, 0)),
            pl.BlockSpec((2, 128), lambda c, i: (0, 0)),
        ],
        out_specs=[
            pl.BlockSpec((2, 128), lambda c, i: (0, 0)),
            pl.BlockSpec((2, 128), lambda c, i: (0, 0)),
        ],
        out_shape=[
            jax.ShapeDtypeStruct((2, 128), jnp.float32),
            jax.ShapeDtypeStruct((2, 128), jnp.float32),
        ],
        scratch_shapes=[
            pltpu.VMEM((1, 128), jnp.float32),
            pltpu.VMEM((1, 128), jnp.float32),
        ],
    )(acc_st, dpt, gam2, bet2)


def _lrelu(v):
    return jnp.where(v >= 0, v, SLOPE * v)


def _mid_body(acca_ref, accb_ref, dp_ref, sc_ref, sh_ref, w2a_ref, w2b_ref,
              g2_ref):
    i = pl.program_id(0)
    dis = _dis_from(dp_ref)
    xa = _lrelu(acca_ref[...] * dis * sc_ref[0:1, :] + sh_ref[0:1, :])
    xb = _lrelu(accb_ref[...] * dis * sc_ref[1:2, :] + sh_ref[1:2, :])
    h2 = (jnp.dot(xa, w2a_ref[...], preferred_element_type=jnp.float32)
          + jnp.dot(xb, w2b_ref[...], preferred_element_type=jnp.float32))
    rid = i * RB + lax.broadcasted_iota(jnp.int32, (RB, 1), 0)
    g2_ref[...] = jnp.where(rid < N, h2 * dis, 0.0)


def _mid(acc_st, dpt, sc1, sh1, w2a, w2b):
    return pl.pallas_call(
        _mid_body,
        grid=(NRB,),
        in_specs=[
            pl.BlockSpec((RB, 128), lambda i: (i, 0)),
            pl.BlockSpec((RB, 128), lambda i: (NRB + i, 0)),
            pl.BlockSpec((RB, 2), lambda i: (i, 0)),
            pl.BlockSpec((2, 128), lambda i: (0, 0)),
            pl.BlockSpec((2, 128), lambda i: (0, 0)),
            pl.BlockSpec((128, D_OUT), lambda i: (0, 0)),
            pl.BlockSpec((128, D_OUT), lambda i: (0, 0)),
        ],
        out_specs=pl.BlockSpec((RB, D_OUT), lambda i: (i, 0)),
        out_shape=jax.ShapeDtypeStruct((NP, D_OUT), jnp.float32),
    )(acc_st, acc_st, dpt, sc1, sh1, w2a, w2b)


def _stats2_body(acca_ref, accb_ref, dp_ref, gam_ref, bet_ref,
                 scale_ref, shift_ref, s_ref, q_ref):
    i = pl.program_id(0)
    dis = _dis_from(dp_ref)
    v = (acca_ref[...] + accb_ref[...]) * dis
    ps = jnp.sum(v, axis=0, keepdims=True)
    pq = jnp.sum(v * v, axis=0, keepdims=True)

    @pl.when(i == 0)
    def _():
        s_ref[...] = jnp.zeros_like(s_ref)
        q_ref[...] = jnp.zeros_like(q_ref)

    s_ref[...] += ps
    q_ref[...] += pq

    @pl.when(i == NRB - 1)
    def _():
        m = s_ref[...] * (1.0 / N)
        var = q_ref[...] * (1.0 / N) - m * m
        sc = gam_ref[...] * lax.rsqrt(var + EPS)
        scale_ref[...] = sc
        shift_ref[...] = bet_ref[...] - m * sc


def _stats2(acc_pair, dpt, gam, bet):
    return pl.pallas_call(
        _stats2_body,
        grid=(NRB,),
        in_specs=[
            pl.BlockSpec((RB, D_OUT), lambda i: (i, 0)),
            pl.BlockSpec((RB, D_OUT), lambda i: (NRB + i, 0)),
            pl.BlockSpec((RB, 2), lambda i: (i, 0)),
            pl.BlockSpec((1, D_OUT), lambda i: (0, 0)),
            pl.BlockSpec((1, D_OUT), lambda i: (0, 0)),
        ],
        out_specs=[
            pl.BlockSpec((1, D_OUT), lambda i: (0, 0)),
            pl.BlockSpec((1, D_OUT), lambda i: (0, 0)),
        ],
        out_shape=[
            jax.ShapeDtypeStruct((1, D_OUT), jnp.float32),
            jax.ShapeDtypeStruct((1, D_OUT), jnp.float32),
        ],
        scratch_shapes=[
            pltpu.VMEM((1, D_OUT), jnp.float32),
            pltpu.VMEM((1, D_OUT), jnp.float32),
        ],
    )(acc_pair, acc_pair, dpt, gam, bet)


def _apply_body(acca_ref, accb_ref, dp_ref, sc_ref, sh_ref, out_ref):
    dis = _dis_from(dp_ref)
    out_ref[...] = ((acca_ref[...] + accb_ref[...]) * dis * sc_ref[...]
                    + sh_ref[...])


def _apply2(acc_pair, dpt, sc2, sh2):
    return pl.pallas_call(
        _apply_body,
        grid=(NRB,),
        in_specs=[
            pl.BlockSpec((RB, D_OUT), lambda i: (i, 0)),
            pl.BlockSpec((RB, D_OUT), lambda i: (NRB + i, 0)),
            pl.BlockSpec((RB, 2), lambda i: (i, 0)),
            pl.BlockSpec((1, D_OUT), lambda i: (0, 0)),
            pl.BlockSpec((1, D_OUT), lambda i: (0, 0)),
        ],
        out_specs=pl.BlockSpec((RB, D_OUT), lambda i: (i, 0)),
        out_shape=jax.ShapeDtypeStruct((N, D_OUT), jnp.float32),
    )(acc_pair, acc_pair, dpt, sc2, sh2)


def kernel(GO_adj, GO_weight, drug_smiles_fea, W1, b1, gamma1, beta1,
           W2, b2, gamma2, beta2):
    row = GO_adj[0].astype(jnp.int32)
    col = GO_adj[1].astype(jnp.int32)
    ew = GO_weight.astype(jnp.float32)
    x = jnp.pad(drug_smiles_fea, ((0, NP - N), (0, 0)))

    e = row.shape[0]
    e_pad = ((e + 4095) // 4096) * 4096
    pad = e_pad - e
    row_p = jnp.pad(row, (0, pad))
    col_p = jnp.pad(col, (0, pad))
    ew_p = jnp.pad(ew, (0, pad))
    rowst = jnp.concatenate([row_p, row_p + NP])

    zeros_n = jnp.zeros((NP,), jnp.float32)
    zeros_nd = jnp.zeros((NP, D_OUT), jnp.float32)
    gam1 = gamma1.reshape(2, 128)
    bet1 = beta1.reshape(2, 128)
    gam2 = gamma2.reshape(1, D_OUT)
    bet2 = beta2.reshape(1, D_OUT)
    w2a = W2[:128]
    w2b = W2[128:]

    deg_pair = _deg(col_p, ew_p, zeros_n, e_pad=e_pad)
    dpt = deg_pair.T

    g1_st = _pre1(x, W1, dpt)
    acc1_st = _scat_featsplit(rowst, col_p, ew_p, g1_st,
                              e_pad=e_pad, dh=128)
    sc1, sh1 = _stats1(acc1_st, dpt, gam1, bet1)
    g2 = _mid(acc1_st, dpt, sc1, sh1, w2a, w2b)
    acc2 = _scat_edgesplit(row_p, col_p, ew_p, g2, zeros_nd,
                           e_pad=e_pad, dh=D_OUT)
    sc2, sh2 = _stats2(acc2, dpt, gam2, bet2)
    out = _apply2(acc2, dpt, sc2, sh2)
    return out


# speedup vs baseline: 8.0451x; 1.4670x over previous
import functools

import jax
import jax.numpy as jnp
from jax import lax
from jax.experimental import pallas as pl
from jax.experimental.pallas import tpu as pltpu
from jax.experimental.pallas import tpu_sc as plsc

N = 10000
NP = 10240
D_IN = 256
D_HID = 256
D_OUT = 128
CHUNK = 128
NBUF = 2
RB = 640
NRB = NP // RB
RPT = NP // 16
EPS = 1e-5
SLOPE = 0.01


@functools.cache
def _mesh():
    return plsc.VectorSubcoreMesh(core_axis_name="c", subcore_axis_name="s")


def _scale_chunk(rows_ref, ew_ref, j, ngrp):
    def scale(g16, carry):
        wvec = ew_ref[j, pl.ds(g16 * 16, 16)]
        for lane in range(16):
            w = wvec[lane]
            kk = g16 * 16 + lane
            for gi in range(ngrp):
                sl = pl.ds(gi * 16, 16)
                rows_ref[kk, sl] = rows_ref[kk, sl] * w
        return carry

    lax.fori_loop(0, CHUNK // 16, scale, 0)


def _edge_pipeline(g_hbm, acc_sh, idx_b, col_b, ew_b, rows, gsems, ssems,
                   nch, ngrp):
    def issue_gather(jn, bn):
        pltpu.async_copy(g_hbm.at[idx_b.at[jn]], rows[bn], gsems[bn])

    def wait_gather(j, b):
        pltpu.make_async_copy(g_hbm.at[idx_b.at[0]], rows[b], gsems[b]).wait()

    def issue_scatter(j, b):
        pltpu.async_copy(rows[b], acc_sh.at[col_b.at[j]], ssems[b], add=True)

    def wait_scatter(b):
        pltpu.make_async_copy(rows[b], acc_sh.at[col_b.at[0]],
                              ssems[b]).wait()

    issue_gather(0, 0)
    issue_gather(1, 1)

    def outer(g, carry):
        for b in range(NBUF):
            j = g * NBUF + b
            wait_gather(j, b)
            _scale_chunk(rows[b], ew_b, j, ngrp)
            issue_scatter(j, b)
            jn = j + NBUF

            @pl.when(jn < nch)
            def _():
                wait_scatter(b)
                issue_gather(jn, b)

        return carry

    lax.fori_loop(0, nch // NBUF, outer, 0)
    for b in range(NBUF):
        wait_scatter(b)


@functools.partial(jax.jit, static_argnames=("e_pad",))
def _deg(col2, ew2, zeros_n, *, e_pad):
    nch = e_pad // 32 // CHUNK

    @functools.partial(
        pl.kernel,
        mesh=_mesh(),
        out_type=jax.ShapeDtypeStruct((2, NP), jnp.float32),
        scratch_types=[
            pltpu.VMEM((e_pad // 32 // CHUNK, CHUNK), jnp.int32),
            pltpu.VMEM((e_pad // 32 // CHUNK, CHUNK), jnp.float32),
            pltpu.VMEM_SHARED((NP,), jnp.float32),
            pltpu.SemaphoreType.DMA,
        ],
    )
    def k(col_hbm, ew_hbm, zeros_hbm, out_hbm, col_b, ew_b, deg_sh, sem):
        c = lax.axis_index("c")
        s = lax.axis_index("s")

        @pl.when(s == 0)
        def _():
            pltpu.sync_copy(zeros_hbm, deg_sh)

        base = c * (e_pad // 2 // CHUNK) + s * nch
        pltpu.sync_copy(col_hbm.at[pl.ds(base, nch)], col_b)
        pltpu.sync_copy(ew_hbm.at[pl.ds(base, nch)], ew_b)
        plsc.subcore_barrier()

        def body(j, carry):
            pltpu.async_copy(ew_b.at[j], deg_sh.at[col_b.at[j]], sem,
                             add=True)
            return carry

        lax.fori_loop(0, nch, body, 0)

        def drain(j, carry):
            pltpu.make_async_copy(ew_b.at[0], deg_sh.at[col_b.at[0]],
                                  sem).wait()
            return carry

        lax.fori_loop(0, nch, drain, 0)
        plsc.subcore_barrier()

        @pl.when(s == 0)
        def _():
            pltpu.sync_copy(deg_sh, out_hbm.at[c])

    return k(col2, ew2, zeros_n)


@functools.partial(jax.jit, static_argnames=("e_pad", "dh"))
def _scat_featsplit(rowst2, col2, ew2, g, *, e_pad, dh):
    nph = 2
    nch = e_pad // 16 // CHUNK // nph
    ngrp = dh // 16

    @functools.partial(
        pl.kernel,
        mesh=_mesh(),
        out_type=jax.ShapeDtypeStruct((2 * NP, dh), jnp.float32),
        scratch_types=(
            [pltpu.VMEM((nch, CHUNK), jnp.int32),
             pltpu.VMEM((nch, CHUNK), jnp.int32),
             pltpu.VMEM((nch, CHUNK), jnp.float32)]
            + [pltpu.VMEM((CHUNK, dh), jnp.float32)] * NBUF
            + [pltpu.VMEM_SHARED((NP, dh), jnp.float32)]
            + [pltpu.SemaphoreType.DMA] * (2 * NBUF)
        ),
    )
    def k(rowst_hbm, col_hbm, ew_hbm, g_hbm, out_hbm,
          idx_b, col_b, ew_b, r0, r1, acc_sh,
          gs0, gs1, ss0, ss1):
        c = lax.axis_index("c")
        s = lax.axis_index("s")
        rows = [r0, r1]
        gsems = [gs0, gs1]
        ssems = [ss0, ss1]
        pltpu.sync_copy(g_hbm.at[pl.ds(c * NP + s * RPT, RPT)],
                        acc_sh.at[pl.ds(s * RPT, RPT)])
        plsc.subcore_barrier()
        for p in range(nph):
            tb = s * (nph * nch) + p * nch
            pltpu.sync_copy(
                rowst_hbm.at[pl.ds(c * (e_pad // CHUNK) + tb, nch)], idx_b)
            pltpu.sync_copy(col_hbm.at[pl.ds(tb, nch)], col_b)
            pltpu.sync_copy(ew_hbm.at[pl.ds(tb, nch)], ew_b)
            _edge_pipeline(g_hbm, acc_sh, idx_b, col_b, ew_b, rows,
                           gsems, ssems, nch, ngrp)
        plsc.subcore_barrier()
        pltpu.sync_copy(acc_sh.at[pl.ds(s * RPT, RPT)],
                        out_hbm.at[pl.ds(c * NP + s * RPT, RPT)])

    return k(rowst2, col2, ew2, g)


@functools.partial(jax.jit, static_argnames=("e_pad", "dh"))
def _scat_edgesplit(row2, col2, ew2, g, zeros_nd, *, e_pad, dh):
    nch = e_pad // 32 // CHUNK
    ngrp = dh // 16

    @functools.partial(
        pl.kernel,
        mesh=_mesh(),
        out_type=jax.ShapeDtypeStruct((2 * NP, dh), jnp.float32),
        scratch_types=(
            [pltpu.VMEM((nch, CHUNK), jnp.int32),
             pltpu.VMEM((nch, CHUNK), jnp.int32),
             pltpu.VMEM((nch, CHUNK), jnp.float32)]
            + [pltpu.VMEM((CHUNK, dh), jnp.float32)] * NBUF
            + [pltpu.VMEM_SHARED((NP, dh), jnp.float32)]
            + [pltpu.SemaphoreType.DMA] * (2 * NBUF)
        ),
    )
    def k(row_hbm, col_hbm, ew_hbm, g_hbm, zeros_hbm, out_hbm,
          idx_b, col_b, ew_b, r0, r1, acc_sh,
          gs0, gs1, ss0, ss1):
        c = lax.axis_index("c")
        s = lax.axis_index("s")
        rows = [r0, r1]
        gsems = [gs0, gs1]
        ssems = [ss0, ss1]

        @pl.when(c == 0)
        def _():
            pltpu.sync_copy(g_hbm.at[pl.ds(s * RPT, RPT)],
                            acc_sh.at[pl.ds(s * RPT, RPT)])

        @pl.when(c != 0)
        def _():
            pltpu.sync_copy(zeros_hbm.at[pl.ds(s * RPT, RPT)],
                            acc_sh.at[pl.ds(s * RPT, RPT)])

        base = c * (e_pad // 2 // CHUNK) + s * nch
        pltpu.sync_copy(row_hbm.at[pl.ds(base, nch)], idx_b)
        pltpu.sync_copy(col_hbm.at[pl.ds(base, nch)], col_b)
        pltpu.sync_copy(ew_hbm.at[pl.ds(base, nch)], ew_b)
        plsc.subcore_barrier()
        _edge_pipeline(g_hbm, acc_sh, idx_b, col_b, ew_b, rows,
                       gsems, ssems, nch, ngrp)
        plsc.subcore_barrier()
        pltpu.sync_copy(acc_sh.at[pl.ds(s * RPT, RPT)],
                        out_hbm.at[pl.ds(c * NP + s * RPT, RPT)])

    return k(row2, col2, ew2, g, zeros_nd)


def _dis_from(dp_ref):
    deg = 1.0 + dp_ref[:, 0:1] + dp_ref[:, 1:2]
    return lax.rsqrt(deg)


def _pre1_body(x_ref, w_ref, dp_ref, g_ref):
    dis = _dis_from(dp_ref)
    h = jnp.dot(x_ref[...], w_ref[...], preferred_element_type=jnp.float32)
    g_ref[...] = h * dis


def _pre1(x, w1, dpt):
    return pl.pallas_call(
        _pre1_body,
        grid=(2, NRB),
        in_specs=[
            pl.BlockSpec((RB, D_IN), lambda c, i: (i, 0)),
            pl.BlockSpec((D_IN, 128), lambda c, i: (0, c)),
            pl.BlockSpec((RB, 2), lambda c, i: (i, 0)),
        ],
        out_specs=pl.BlockSpec((RB, 128), lambda c, i: (c * NRB + i, 0)),
        out_shape=jax.ShapeDtypeStruct((2 * NP, 128), jnp.float32),
    )(x, w1, dpt)


def _stats1_body(acc_ref, dp_ref, gam_ref, bet_ref, scale_ref, shift_ref,
                 s_ref, q_ref):
    c = pl.program_id(0)
    i = pl.program_id(1)
    dis = _dis_from(dp_ref)
    v = acc_ref[...] * dis
    ps = jnp.sum(v, axis=0, keepdims=True)
    pq = jnp.sum(v * v, axis=0, keepdims=True)

    @pl.when(i == 0)
    def _():
        s_ref[...] = jnp.zeros_like(s_ref)
        q_ref[...] = jnp.zeros_like(q_ref)

    s_ref[...] += ps
    q_ref[...] += pq

    @pl.when(i == NRB - 1)
    def _():
        m = s_ref[...] * (1.0 / N)
        var = q_ref[...] * (1.0 / N) - m * m
        sc = gam_ref[pl.ds(c, 1), :] * lax.rsqrt(var + EPS)
        scale_ref[pl.ds(c, 1), :] = sc
        shift_ref[pl.ds(c, 1), :] = bet_ref[pl.ds(c, 1), :] - m * sc


def _stats1(acc_st, dpt, gam2, bet2):
    return pl.pallas_call(
        _stats1_body,
        grid=(2, NRB),
        in_specs=[
            pl.BlockSpec((RB, 128), lambda c, i: (c * NRB + i, 0)),
            pl.BlockSpec((RB, 2), lambda c, i: (i, 0)),
            pl.BlockSpec((2, 128), lambda c, i: (0, 0)),
            pl.BlockSpec((2, 128), lambda c, i: (0, 0)),
        ],
        out_specs=[
            pl.BlockSpec((2, 128), lambda c, i: (0, 0)),
            pl.BlockSpec((2, 128), lambda c, i: (0, 0)),
        ],
        out_shape=[
            jax.ShapeDtypeStruct((2, 128), jnp.float32),
            jax.ShapeDtypeStruct((2, 128), jnp.float32),
        ],
        scratch_shapes=[
            pltpu.VMEM((1, 128), jnp.float32),
            pltpu.VMEM((1, 128), jnp.float32),
        ],
    )(acc_st, dpt, gam2, bet2)


def _lrelu(v):
    return jnp.where(v >= 0, v, SLOPE * v)


def _mid_body(acca_ref, accb_ref, dp_ref, sc_ref, sh_ref, w2a_ref, w2b_ref,
              g2_ref):
    i = pl.program_id(0)
    dis = _dis_from(dp_ref)
    xa = _lrelu(acca_ref[...] * dis * sc_ref[0:1, :] + sh_ref[0:1, :])
    xb = _lrelu(accb_ref[...] * dis * sc_ref[1:2, :] + sh_ref[1:2, :])
    h2 = (jnp.dot(xa, w2a_ref[...], preferred_element_type=jnp.float32)
          + jnp.dot(xb, w2b_ref[...], preferred_element_type=jnp.float32))
    rid = i * RB + lax.broadcasted_iota(jnp.int32, (RB, 1), 0)
    g2_ref[...] = jnp.where(rid < N, h2 * dis, 0.0)


def _mid(acc_st, dpt, sc1, sh1, w2a, w2b):
    return pl.pallas_call(
        _mid_body,
        grid=(NRB,),
        in_specs=[
            pl.BlockSpec((RB, 128), lambda i: (i, 0)),
            pl.BlockSpec((RB, 128), lambda i: (NRB + i, 0)),
            pl.BlockSpec((RB, 2), lambda i: (i, 0)),
            pl.BlockSpec((2, 128), lambda i: (0, 0)),
            pl.BlockSpec((2, 128), lambda i: (0, 0)),
            pl.BlockSpec((128, D_OUT), lambda i: (0, 0)),
            pl.BlockSpec((128, D_OUT), lambda i: (0, 0)),
        ],
        out_specs=pl.BlockSpec((RB, D_OUT), lambda i: (i, 0)),
        out_shape=jax.ShapeDtypeStruct((NP, D_OUT), jnp.float32),
    )(acc_st, acc_st, dpt, sc1, sh1, w2a, w2b)


def _stats2_body(acca_ref, accb_ref, dp_ref, gam_ref, bet_ref,
                 scale_ref, shift_ref, s_ref, q_ref):
    i = pl.program_id(0)
    dis = _dis_from(dp_ref)
    v = (acca_ref[...] + accb_ref[...]) * dis
    ps = jnp.sum(v, axis=0, keepdims=True)
    pq = jnp.sum(v * v, axis=0, keepdims=True)

    @pl.when(i == 0)
    def _():
        s_ref[...] = jnp.zeros_like(s_ref)
        q_ref[...] = jnp.zeros_like(q_ref)

    s_ref[...] += ps
    q_ref[...] += pq

    @pl.when(i == NRB - 1)
    def _():
        m = s_ref[...] * (1.0 / N)
        var = q_ref[...] * (1.0 / N) - m * m
        sc = gam_ref[...] * lax.rsqrt(var + EPS)
        scale_ref[...] = sc
        shift_ref[...] = bet_ref[...] - m * sc


def _stats2(acc_pair, dpt, gam, bet):
    return pl.pallas_call(
        _stats2_body,
        grid=(NRB,),
        in_specs=[
            pl.BlockSpec((RB, D_OUT), lambda i: (i, 0)),
            pl.BlockSpec((RB, D_OUT), lambda i: (NRB + i, 0)),
            pl.BlockSpec((RB, 2), lambda i: (i, 0)),
            pl.BlockSpec((1, D_OUT), lambda i: (0, 0)),
            pl.BlockSpec((1, D_OUT), lambda i: (0, 0)),
        ],
        out_specs=[
            pl.BlockSpec((1, D_OUT), lambda i: (0, 0)),
            pl.BlockSpec((1, D_OUT), lambda i: (0, 0)),
        ],
        out_shape=[
            jax.ShapeDtypeStruct((1, D_OUT), jnp.float32),
            jax.ShapeDtypeStruct((1, D_OUT), jnp.float32),
        ],
        scratch_shapes=[
            pltpu.VMEM((1, D_OUT), jnp.float32),
            pltpu.VMEM((1, D_OUT), jnp.float32),
        ],
    )(acc_pair, acc_pair, dpt, gam, bet)


def _apply_body(acca_ref, accb_ref, dp_ref, sc_ref, sh_ref, out_ref):
    dis = _dis_from(dp_ref)
    out_ref[...] = ((acca_ref[...] + accb_ref[...]) * dis * sc_ref[...]
                    + sh_ref[...])


def _apply2(acc_pair, dpt, sc2, sh2):
    return pl.pallas_call(
        _apply_body,
        grid=(NRB,),
        in_specs=[
            pl.BlockSpec((RB, D_OUT), lambda i: (i, 0)),
            pl.BlockSpec((RB, D_OUT), lambda i: (NRB + i, 0)),
            pl.BlockSpec((RB, 2), lambda i: (i, 0)),
            pl.BlockSpec((1, D_OUT), lambda i: (0, 0)),
            pl.BlockSpec((1, D_OUT), lambda i: (0, 0)),
        ],
        out_specs=pl.BlockSpec((RB, D_OUT), lambda i: (i, 0)),
        out_shape=jax.ShapeDtypeStruct((N, D_OUT), jnp.float32),
    )(acc_pair, acc_pair, dpt, sc2, sh2)


def kernel(GO_adj, GO_weight, drug_smiles_fea, W1, b1, gamma1, beta1,
           W2, b2, gamma2, beta2):
    row = GO_adj[0].astype(jnp.int32)
    col = GO_adj[1].astype(jnp.int32)
    ew = GO_weight.astype(jnp.float32)
    x = jnp.pad(drug_smiles_fea, ((0, NP - N), (0, 0)))

    e = row.shape[0]
    e_pad = ((e + 4095) // 4096) * 4096
    pad = e_pad - e
    row_p = jnp.pad(row, (0, pad))
    col_p = jnp.pad(col, (0, pad))
    ew_p = jnp.pad(ew, (0, pad))
    row2 = row_p.reshape(-1, CHUNK)
    col2 = col_p.reshape(-1, CHUNK)
    ew2 = ew_p.reshape(-1, CHUNK)
    rowst2 = jnp.concatenate([row_p, row_p + NP]).reshape(-1, CHUNK)

    zeros_n = jnp.zeros((NP,), jnp.float32)
    zeros_nd = jnp.zeros((NP, D_OUT), jnp.float32)
    gam1 = gamma1.reshape(2, 128)
    bet1 = beta1.reshape(2, 128)
    gam2 = gamma2.reshape(1, D_OUT)
    bet2 = beta2.reshape(1, D_OUT)
    w2a = W2[:128]
    w2b = W2[128:]

    deg_pair = _deg(col2, ew2, zeros_n, e_pad=e_pad)
    dpt = deg_pair.T

    g1_st = _pre1(x, W1, dpt)
    acc1_st = _scat_featsplit(rowst2, col2, ew2, g1_st,
                              e_pad=e_pad, dh=128)
    sc1, sh1 = _stats1(acc1_st, dpt, gam1, bet1)
    g2 = _mid(acc1_st, dpt, sc1, sh1, w2a, w2b)
    acc2 = _scat_edgesplit(row2, col2, ew2, g2, zeros_nd,
                           e_pad=e_pad, dh=D_OUT)
    sc2, sh2 = _stats2(acc2, dpt, gam2, bet2)
    out = _apply2(acc2, dpt, sc2, sh2)
    return out
